# baseline (speedup 1.0000x reference)
import functools

import jax
import jax.numpy as jnp
from jax import lax
from jax.experimental import pallas as pl
from jax.experimental.pallas import tpu as pltpu, tpu_sc as plsc

_INFO = plsc.get_sparse_core_info()
_NC = _INFO.num_cores
_NS = _INFO.num_subcores
_NW = _NC * _NS
_L = _INFO.num_lanes

_B = 16384
_D = 64
_V = 1000000
_PER_W = _B // _NW
_CH = 128
_NCHUNK = _PER_W // _CH

_R = 8192
_G = -(-_V // _R)


def _matvec_body(u_idx_ref, wu_ref, ww_ref, wk_ref, sw_ref, sk_ref):
    u = wu_ref[u_idx_ref[0] % 8, :]
    u_lo = u[:_D]
    u_hi = u[_D:]
    sw_ref[...] = jax.lax.dot_general(
        ww_ref[...], u_lo, (((1,), (0,)), ((), ())),
        precision=lax.Precision.HIGHEST,
        preferred_element_type=jnp.float32)
    sk_ref[...] = jax.lax.dot_general(
        wk_ref[...], u_hi, (((1,), (0,)), ((), ())),
        precision=lax.Precision.HIGHEST,
        preferred_element_type=jnp.float32)


_matvec_tc = pl.pallas_call(
    _matvec_body,
    grid_spec=pltpu.PrefetchScalarGridSpec(
        num_scalar_prefetch=1,
        grid=(_G,),
        in_specs=[
            pl.BlockSpec((8, 2 * _D), lambda i, uref: (uref[0] // 8, 0)),
            pl.BlockSpec((_R, _D), lambda i, uref: (i, 0)),
            pl.BlockSpec((_R, _D), lambda i, uref: (i, 0)),
        ],
        out_specs=[
            pl.BlockSpec((_R,), lambda i, uref: (i,)),
            pl.BlockSpec((_R,), lambda i, uref: (i,)),
        ],
    ),
    out_shape=[
        jax.ShapeDtypeStruct((_V,), jnp.float32),
        jax.ShapeDtypeStruct((_V,), jnp.float32),
    ],
)


@functools.partial(
    pl.kernel,
    mesh=plsc.VectorSubcoreMesh(core_axis_name="c", subcore_axis_name="s"),
    compiler_params=pltpu.CompilerParams(
        needs_layout_passes=False, use_tc_tiling_on_sc=False),
    out_type=jax.ShapeDtypeStruct((_NW, _L), jnp.float32),
    scratch_types=[
        pltpu.VMEM((4, _PER_W), jnp.int32),
        pltpu.VMEM((_PER_W,), jnp.float32),
        pltpu.VMEM((_PER_W,), jnp.float32),
        pltpu.VMEM((_PER_W,), jnp.float32),
        pltpu.VMEM((_PER_W,), jnp.float32),
        pltpu.VMEM((_L,), jnp.float32),
        pltpu.SemaphoreType.DMA,
    ],
)
def _score_sc(data_hbm, sw_hbm, sk_hbm, out_hbm,
              idx_all, gwp, gkp, gwn, gkn, sum_v, sem):
    wid = lax.axis_index("s") * _NC + lax.axis_index("c")
    base = wid * _PER_W

    pltpu.sync_copy(data_hbm.at[pl.ds(1, 4), pl.ds(base, _PER_W)], idx_all)

    cps = []
    for c in range(_NCHUNK):
        sl = pl.ds(c * _CH, _CH)
        cps += [
            pltpu.async_copy(sw_hbm.at[idx_all.at[1, sl]], gwp.at[sl], sem),
            pltpu.async_copy(sk_hbm.at[idx_all.at[0, sl]], gkp.at[sl], sem),
            pltpu.async_copy(sw_hbm.at[idx_all.at[3, sl]], gwn.at[sl], sem),
            pltpu.async_copy(sk_hbm.at[idx_all.at[2, sl]], gkn.at[sl], sem),
        ]
    for cp in cps:
        cp.wait()

    one = jnp.float32(1.0)
    two = jnp.float32(2.0)

    def sigmoid(x):
        y = one + jnp.exp(-x)
        r = one / y
        r = r * (two - y * r)
        return r * (two - y * r)

    def jbody(j, total):
        sl = pl.ds(j * _L, _L)
        pos = gwp[sl] + gkp[sl]
        neg = gwn[sl] + gkn[sl]
        return total + sigmoid(pos) - sigmoid(neg)

    total = lax.fori_loop(0, _PER_W // _L, jbody,
                          jnp.zeros((_L,), jnp.float32))

    sum_v[...] = total
    pltpu.sync_copy(sum_v, out_hbm.at[wid])


def kernel(data, W_user, W_writer, W_keywd):
    data = data.astype(jnp.int32)
    u_idx = data[0, 0:1]
    s_w, s_k = _matvec_tc(u_idx, W_user, W_writer, W_keywd)
    partials = _score_sc(data, s_w, s_k)
    return jnp.sum(partials)


# speedup vs baseline: 2.1914x; 2.1914x over previous
import functools

import jax
import jax.numpy as jnp
from jax import lax
from jax.experimental import pallas as pl
from jax.experimental.pallas import tpu as pltpu, tpu_sc as plsc

_INFO = plsc.get_sparse_core_info()
_NC = _INFO.num_cores
_NS = _INFO.num_subcores
_NW = _NC * _NS
_L = _INFO.num_lanes

_B = 16384
_D = 64
_V = 1000000
_PER_W = _B // _NW
_T = 16
_NCHUNK = _PER_W // _T


@functools.partial(
    pl.kernel,
    mesh=plsc.VectorSubcoreMesh(core_axis_name="c", subcore_axis_name="s"),
    compiler_params=pltpu.CompilerParams(
        needs_layout_passes=False, use_tc_tiling_on_sc=True),
    out_type=jax.ShapeDtypeStruct((_NW, _L), jnp.float32),
    scratch_types=[
        pltpu.VMEM((4, _PER_W), jnp.int32),
        pltpu.VMEM((_L,), jnp.int32),
        pltpu.VMEM((1, 2 * _D), jnp.float32),
        pltpu.VMEM((_D, _L), jnp.float32),
        pltpu.VMEM((_D, _L), jnp.float32),
        pltpu.VMEM((2, _T, _D), jnp.float32),
        pltpu.VMEM((2, _T, _D), jnp.float32),
        pltpu.VMEM((2, _T, _D), jnp.float32),
        pltpu.VMEM((2, _T, _D), jnp.float32),
        pltpu.VMEM((_L,), jnp.float32),
        pltpu.SemaphoreType.DMA,
        pltpu.SemaphoreType.DMA,
        pltpu.SemaphoreType.DMA,
    ],
)
def _gmf_sc(data_hbm, wu_hbm, ww_hbm, wk_hbm, out_hbm,
            idx_all, idx_u, u_row, u_rot_lo, u_rot_hi,
            w_r, kw_r, nw_r, nkw_r, sum_v, sem0, sem1, sem_s):
    wid = lax.axis_index("s") * _NC + lax.axis_index("c")
    base = wid * _PER_W

    iota16 = lax.iota(jnp.int32, _L)

    pltpu.async_copy(
        data_hbm.at[pl.ds(1, 4), pl.ds(base, _PER_W)], idx_all, sem_s).wait()
    pltpu.async_copy(data_hbm.at[0, pl.ds(0, _L)], idx_u, sem_s).wait()
    u0 = idx_u[...]
    pltpu.async_copy(wu_hbm.at[pl.ds(u0[0], 1), :], u_row, sem_s).wait()

    def ubody(d, _):
        colv = jnp.bitwise_and(iota16 + d, _D - 1)
        z16 = jnp.zeros((_L,), jnp.int32)
        u_rot_lo[d, :] = plsc.load_gather(u_row, [z16, colv])
        u_rot_hi[d, :] = plsc.load_gather(u_row, [z16, colv + _D])
        return 0
    lax.fori_loop(0, _D, ubody, 0)

    tables = (ww_hbm, wk_hbm, ww_hbm, wk_hbm)
    data_rows = (1, 0, 3, 2)

    def issue(c, bufs, sem):
        for tbl, r, dest in zip(tables, data_rows, bufs):
            rows = idx_all[r, pl.ds(c * _T, _T)]
            for e in range(_T):
                pltpu.async_copy(
                    tbl.at[pl.ds(rows[e], 1), :],
                    dest.at[pl.ds(e, 1), :], sem)

    def drain(bufs, sem):
        for tbl, dest in zip(tables, bufs):
            pltpu.make_async_copy(
                tbl.at[pl.ds(0, _T), :], dest, sem).wait()

    one = jnp.float32(1.0)
    two = jnp.float32(2.0)

    def sigmoid(x):
        y = one + jnp.exp(-x)
        r = one / y
        r = r * (two - y * r)
        return r * (two - y * r)

    def compute(bufs):
        def dbody(d, carry):
            pos, neg = carry
            colv = jnp.bitwise_and(iota16 + d, _D - 1)
            ul = u_rot_lo[d, :]
            uh = u_rot_hi[d, :]
            pos = (pos
                   + plsc.load_gather(bufs[0], [iota16, colv]) * ul
                   + plsc.load_gather(bufs[1], [iota16, colv]) * uh)
            neg = (neg
                   + plsc.load_gather(bufs[2], [iota16, colv]) * ul
                   + plsc.load_gather(bufs[3], [iota16, colv]) * uh)
            return pos, neg

        zero = jnp.zeros((_L,), jnp.float32)
        pos, neg = lax.fori_loop(0, _D, dbody, (zero, zero))
        return sigmoid(pos) - sigmoid(neg)

    buf0 = (w_r.at[0], kw_r.at[0], nw_r.at[0], nkw_r.at[0])
    buf1 = (w_r.at[1], kw_r.at[1], nw_r.at[1], nkw_r.at[1])

    issue(0, buf0, sem0)

    def pair_body(p, total):
        c = p * 2
        issue(c + 1, buf1, sem1)
        drain(buf0, sem0)
        total = total + compute(buf0)

        @pl.when(p < _NCHUNK // 2 - 1)
        def _():
            issue(c + 2, buf0, sem0)

        drain(buf1, sem1)
        return total + compute(buf1)

    total = lax.fori_loop(0, _NCHUNK // 2, pair_body,
                          jnp.zeros((_L,), jnp.float32))

    sum_v[...] = total
    pltpu.sync_copy(sum_v, out_hbm.at[wid])


def kernel(data, W_user, W_writer, W_keywd):
    data = data.astype(jnp.int32)
    partials = _gmf_sc(data, W_user, W_writer, W_keywd)
    return jnp.sum(partials)


# speedup vs baseline: 2.1958x; 1.0020x over previous
import functools

import jax
import jax.numpy as jnp
from jax import lax
from jax.experimental import pallas as pl
from jax.experimental.pallas import tpu as pltpu, tpu_sc as plsc

_INFO = plsc.get_sparse_core_info()
_NC = _INFO.num_cores
_NS = _INFO.num_subcores
_NW = _NC * _NS
_L = _INFO.num_lanes

_B = 16384
_D = 64
_V = 1000000
_PER_W = _B // _NW
_T = 16
_NCHUNK = _PER_W // _T


@functools.partial(
    pl.kernel,
    mesh=plsc.VectorSubcoreMesh(core_axis_name="c", subcore_axis_name="s"),
    compiler_params=pltpu.CompilerParams(
        needs_layout_passes=False, use_tc_tiling_on_sc=True,
        skip_device_barrier=True),
    out_type=jax.ShapeDtypeStruct((_NW, _L), jnp.float32),
    scratch_types=[
        pltpu.VMEM((4, _PER_W), jnp.int32),
        pltpu.VMEM((_L,), jnp.int32),
        pltpu.VMEM((1, 2 * _D), jnp.float32),
        pltpu.VMEM((_D, _L), jnp.float32),
        pltpu.VMEM((_D, _L), jnp.float32),
        pltpu.VMEM((2, _T, _D), jnp.float32),
        pltpu.VMEM((2, _T, _D), jnp.float32),
        pltpu.VMEM((2, _T, _D), jnp.float32),
        pltpu.VMEM((2, _T, _D), jnp.float32),
        pltpu.VMEM((_L,), jnp.float32),
        pltpu.SemaphoreType.DMA,
        pltpu.SemaphoreType.DMA,
        pltpu.SemaphoreType.DMA,
    ],
)
def _gmf_sc(data_hbm, wu_hbm, ww_hbm, wk_hbm, out_hbm,
            idx_all, idx_u, u_row, u_rot_lo, u_rot_hi,
            w_r, kw_r, nw_r, nkw_r, sum_v, sem0, sem1, sem_s):
    wid = lax.axis_index("s") * _NC + lax.axis_index("c")
    base = wid * _PER_W

    iota16 = lax.iota(jnp.int32, _L)

    pltpu.async_copy(
        data_hbm.at[pl.ds(1, 4), pl.ds(base, _PER_W)], idx_all, sem_s).wait()
    pltpu.async_copy(data_hbm.at[0, pl.ds(0, _L)], idx_u, sem_s).wait()
    u0 = idx_u[...]
    pltpu.async_copy(wu_hbm.at[pl.ds(u0[0], 1), :], u_row, sem_s).wait()

    def ubody(d, _):
        colv = jnp.bitwise_and(iota16 + d, _D - 1)
        z16 = jnp.zeros((_L,), jnp.int32)
        u_rot_lo[d, :] = plsc.load_gather(u_row, [z16, colv])
        u_rot_hi[d, :] = plsc.load_gather(u_row, [z16, colv + _D])
        return 0
    lax.fori_loop(0, _D, ubody, 0)

    tables = (ww_hbm, wk_hbm, ww_hbm, wk_hbm)
    data_rows = (1, 0, 3, 2)

    def issue(c, bufs, sem):
        for tbl, r, dest in zip(tables, data_rows, bufs):
            rows = idx_all[r, pl.ds(c * _T, _T)]
            for e in range(_T):
                pltpu.async_copy(
                    tbl.at[pl.ds(rows[e], 1), :],
                    dest.at[pl.ds(e, 1), :], sem)

    def drain(bufs, sem):
        for tbl, dest in zip(tables, bufs):
            pltpu.make_async_copy(
                tbl.at[pl.ds(0, _T), :], dest, sem).wait()

    one = jnp.float32(1.0)
    two = jnp.float32(2.0)

    def sigmoid(x):
        y = one + jnp.exp(-x)
        r = one / y
        r = r * (two - y * r)
        return r * (two - y * r)

    def compute(bufs):
        def dbody(d, carry):
            pos, neg = carry
            colv = jnp.bitwise_and(iota16 + d, _D - 1)
            ul = u_rot_lo[d, :]
            uh = u_rot_hi[d, :]
            pos = (pos
                   + plsc.load_gather(bufs[0], [iota16, colv]) * ul
                   + plsc.load_gather(bufs[1], [iota16, colv]) * uh)
            neg = (neg
                   + plsc.load_gather(bufs[2], [iota16, colv]) * ul
                   + plsc.load_gather(bufs[3], [iota16, colv]) * uh)
            return pos, neg

        zero = jnp.zeros((_L,), jnp.float32)
        pos, neg = lax.fori_loop(0, _D, dbody, (zero, zero))
        return sigmoid(pos) - sigmoid(neg)

    buf0 = (w_r.at[0], kw_r.at[0], nw_r.at[0], nkw_r.at[0])
    buf1 = (w_r.at[1], kw_r.at[1], nw_r.at[1], nkw_r.at[1])

    issue(0, buf0, sem0)

    def pair_body(p, total):
        c = p * 2
        issue(c + 1, buf1, sem1)
        drain(buf0, sem0)
        total = total + compute(buf0)

        @pl.when(p < _NCHUNK // 2 - 1)
        def _():
            issue(c + 2, buf0, sem0)

        drain(buf1, sem1)
        return total + compute(buf1)

    total = lax.fori_loop(0, _NCHUNK // 2, pair_body,
                          jnp.zeros((_L,), jnp.float32))

    sum_v[...] = total
    pltpu.sync_copy(sum_v, out_hbm.at[wid])


def kernel(data, W_user, W_writer, W_keywd):
    data = data.astype(jnp.int32)
    partials = _gmf_sc(data, W_user, W_writer, W_keywd)
    return jnp.sum(partials)


# speedup vs baseline: 7.4723x; 3.4029x over previous
import functools

import jax
import jax.numpy as jnp
from jax import lax
from jax.experimental import pallas as pl
from jax.experimental.pallas import tpu as pltpu, tpu_sc as plsc

_INFO = plsc.get_sparse_core_info()
_NC = _INFO.num_cores
_NS = _INFO.num_subcores
_NW = _NC * _NS
_L = _INFO.num_lanes

_B = 16384
_D = 64
_V = 1000000
_PER_W = _B // _NW
_CH = 128
_NCHUNK = _PER_W // _CH

_C = 8192
_G = -(-_V // _C)


def _matvec_body(u_idx_ref, wu_ref, wtw_ref, wtk_ref, sw_ref, sk_ref):
    u = wu_ref[u_idx_ref[0] % 8, :]
    u_lo = u[:_D][:, None]
    u_hi = u[_D:][:, None]
    sw_ref[...] = jnp.sum(wtw_ref[...] * u_lo, axis=0)
    sk_ref[...] = jnp.sum(wtk_ref[...] * u_hi, axis=0)


_matvec_tc = pl.pallas_call(
    _matvec_body,
    grid_spec=pltpu.PrefetchScalarGridSpec(
        num_scalar_prefetch=1,
        grid=(_G,),
        in_specs=[
            pl.BlockSpec((8, 2 * _D), lambda i, uref: (uref[0] // 8, 0)),
            pl.BlockSpec((_D, _C), lambda i, uref: (0, i)),
            pl.BlockSpec((_D, _C), lambda i, uref: (0, i)),
        ],
        out_specs=[
            pl.BlockSpec((_C,), lambda i, uref: (i,)),
            pl.BlockSpec((_C,), lambda i, uref: (i,)),
        ],
    ),
    out_shape=[
        jax.ShapeDtypeStruct((_V,), jnp.float32),
        jax.ShapeDtypeStruct((_V,), jnp.float32),
    ],
)


@functools.partial(
    pl.kernel,
    mesh=plsc.VectorSubcoreMesh(core_axis_name="c", subcore_axis_name="s"),
    compiler_params=pltpu.CompilerParams(
        needs_layout_passes=False, use_tc_tiling_on_sc=False),
    out_type=jax.ShapeDtypeStruct((_NW, _L), jnp.float32),
    scratch_types=[
        pltpu.VMEM((4, _PER_W), jnp.int32),
        pltpu.VMEM((_PER_W,), jnp.float32),
        pltpu.VMEM((_PER_W,), jnp.float32),
        pltpu.VMEM((_PER_W,), jnp.float32),
        pltpu.VMEM((_PER_W,), jnp.float32),
        pltpu.VMEM((_L,), jnp.float32),
        pltpu.SemaphoreType.DMA,
    ],
)
def _score_sc(data_hbm, sw_hbm, sk_hbm, out_hbm,
              idx_all, gwp, gkp, gwn, gkn, sum_v, sem):
    wid = lax.axis_index("s") * _NC + lax.axis_index("c")
    base = wid * _PER_W

    pltpu.sync_copy(data_hbm.at[pl.ds(1, 4), pl.ds(base, _PER_W)], idx_all)

    cps = []
    for c in range(_NCHUNK):
        sl = pl.ds(c * _CH, _CH)
        cps += [
            pltpu.async_copy(sw_hbm.at[idx_all.at[1, sl]], gwp.at[sl], sem),
            pltpu.async_copy(sk_hbm.at[idx_all.at[0, sl]], gkp.at[sl], sem),
            pltpu.async_copy(sw_hbm.at[idx_all.at[3, sl]], gwn.at[sl], sem),
            pltpu.async_copy(sk_hbm.at[idx_all.at[2, sl]], gkn.at[sl], sem),
        ]
    for cp in cps:
        cp.wait()

    one = jnp.float32(1.0)
    two = jnp.float32(2.0)

    def sigmoid(x):
        y = one + jnp.exp(-x)
        r = one / y
        r = r * (two - y * r)
        return r * (two - y * r)

    def jbody(j, total):
        sl = pl.ds(j * _L, _L)
        pos = gwp[sl] + gkp[sl]
        neg = gwn[sl] + gkn[sl]
        return total + sigmoid(pos) - sigmoid(neg)

    total = lax.fori_loop(0, _PER_W // _L, jbody,
                          jnp.zeros((_L,), jnp.float32))

    sum_v[...] = total
    pltpu.sync_copy(sum_v, out_hbm.at[wid])


def kernel(data, W_user, W_writer, W_keywd):
    data = data.astype(jnp.int32)
    u_idx = data[0, 0:1]
    s_w, s_k = _matvec_tc(u_idx, W_user, W_writer.T, W_keywd.T)
    partials = _score_sc(data, s_w, s_k)
    return jnp.sum(partials)


# speedup vs baseline: 8.7400x; 1.1697x over previous
import functools

import jax
import jax.numpy as jnp
from jax import lax
from jax.experimental import pallas as pl
from jax.experimental.pallas import tpu as pltpu, tpu_sc as plsc

_INFO = plsc.get_sparse_core_info()
_NC = _INFO.num_cores
_NS = _INFO.num_subcores
_NW = _NC * _NS
_L = _INFO.num_lanes

_B = 16384
_D = 64
_V = 1000000
_PER_W = _B // _NW
_CH = 128
_NCHUNK = _PER_W // _CH

_C = 16384
_G = -(-_V // _C)


def _matvec_body(u_idx_ref, wu_ref, wtw_ref, wtk_ref, sw_ref, sk_ref):
    u = wu_ref[u_idx_ref[0] % 8, :]
    u_lo = u[:_D][:, None]
    u_hi = u[_D:][:, None]
    sw_ref[...] = jnp.sum(wtw_ref[...] * u_lo, axis=0)
    sk_ref[...] = jnp.sum(wtk_ref[...] * u_hi, axis=0)


_matvec_tc = pl.pallas_call(
    _matvec_body,
    grid_spec=pltpu.PrefetchScalarGridSpec(
        num_scalar_prefetch=1,
        grid=(_G,),
        in_specs=[
            pl.BlockSpec((8, 2 * _D), lambda i, uref: (uref[0] // 8, 0)),
            pl.BlockSpec((_D, _C), lambda i, uref: (0, i)),
            pl.BlockSpec((_D, _C), lambda i, uref: (0, i)),
        ],
        out_specs=[
            pl.BlockSpec((_C,), lambda i, uref: (i,)),
            pl.BlockSpec((_C,), lambda i, uref: (i,)),
        ],
    ),
    out_shape=[
        jax.ShapeDtypeStruct((_V,), jnp.float32),
        jax.ShapeDtypeStruct((_V,), jnp.float32),
    ],
)


@functools.partial(
    pl.kernel,
    mesh=plsc.VectorSubcoreMesh(core_axis_name="c", subcore_axis_name="s"),
    compiler_params=pltpu.CompilerParams(
        needs_layout_passes=False, use_tc_tiling_on_sc=False),
    out_type=jax.ShapeDtypeStruct((_NW, _L), jnp.float32),
    scratch_types=[
        pltpu.VMEM((4, _PER_W), jnp.int32),
        pltpu.VMEM((_PER_W,), jnp.float32),
        pltpu.VMEM((_PER_W,), jnp.float32),
        pltpu.VMEM((_PER_W,), jnp.float32),
        pltpu.VMEM((_PER_W,), jnp.float32),
        pltpu.VMEM((_L,), jnp.float32),
        pltpu.SemaphoreType.DMA,
    ],
)
def _score_sc(data_hbm, sw_hbm, sk_hbm, out_hbm,
              idx_all, gwp, gkp, gwn, gkn, sum_v, sem):
    wid = lax.axis_index("s") * _NC + lax.axis_index("c")
    base = wid * _PER_W

    pltpu.sync_copy(data_hbm.at[pl.ds(1, 4), pl.ds(base, _PER_W)], idx_all)

    cps = []
    for c in range(_NCHUNK):
        sl = pl.ds(c * _CH, _CH)
        cps += [
            pltpu.async_copy(sw_hbm.at[idx_all.at[1, sl]], gwp.at[sl], sem),
            pltpu.async_copy(sk_hbm.at[idx_all.at[0, sl]], gkp.at[sl], sem),
            pltpu.async_copy(sw_hbm.at[idx_all.at[3, sl]], gwn.at[sl], sem),
            pltpu.async_copy(sk_hbm.at[idx_all.at[2, sl]], gkn.at[sl], sem),
        ]
    for cp in cps:
        cp.wait()

    one = jnp.float32(1.0)
    two = jnp.float32(2.0)

    def sigmoid(x):
        y = one + jnp.exp(-x)
        r = one / y
        r = r * (two - y * r)
        return r * (two - y * r)

    def jbody(j, total):
        sl = pl.ds(j * _L, _L)
        pos = gwp[sl] + gkp[sl]
        neg = gwn[sl] + gkn[sl]
        return total + sigmoid(pos) - sigmoid(neg)

    total = lax.fori_loop(0, _PER_W // _L, jbody,
                          jnp.zeros((_L,), jnp.float32))

    sum_v[...] = total
    pltpu.sync_copy(sum_v, out_hbm.at[wid])


def kernel(data, W_user, W_writer, W_keywd):
    data = data.astype(jnp.int32)
    u_idx = data[0, 0:1]
    s_w, s_k = _matvec_tc(u_idx, W_user, W_writer.T, W_keywd.T)
    partials = _score_sc(data, s_w, s_k)
    return jnp.sum(partials)
